# baseline (speedup 1.0000x reference)
import math

import jax
import jax.numpy as jnp
from jax import lax
from jax.experimental import pallas as pl
from jax.experimental.pallas import tpu as pltpu
from jax.experimental.pallas import tpu_sc as plsc

D = 1024
NC = 2
NS = 16
NW = NC * NS
BATCH = 4
SEQ = 4096
TOTAL = BATCH * SEQ
PER_W = TOTAL // NW
WPR = SEQ // PER_W
CH = 16
NCH = PER_W // CH
NSLOT = 7
LEAD = 5
LANES = 16
VPR = D // LANES
SCALE = math.sqrt(D)


def _scale_buf(buf):
    def row(r, carry):
        def grp(j, carry2):
            sl = pl.ds(j * LANES, LANES)
            buf[r, sl] = buf[r, sl] * SCALE
            return carry2

        return lax.fori_loop(0, VPR, grp, carry, unroll=8)

    lax.fori_loop(0, CH, row, 0, unroll=False)


def _body(x_hbm, table_hbm, out_hbm, idx_v, buf, sg, so):
    wid = lax.axis_index("s") * NC + lax.axis_index("c")
    pltpu.sync_copy(
        x_hbm.at[wid // WPR, pl.ds((wid % WPR) * PER_W, PER_W)], idx_v)

    for n in range(LEAD):
        pltpu.async_copy(
            table_hbm.at[idx_v.at[pl.ds(n * CH, CH)]], buf.at[n], sg)

    def step(c, carry):
        n = c + LEAD
        bn = lax.rem(n, NSLOT)

        @pl.when(n < NCH)
        def _():
            @pl.when(c >= NSLOT - LEAD)
            def _():
                pltpu.make_async_copy(
                    buf.at[0], out_hbm.at[wid, 0], so).wait()

            pltpu.async_copy(
                table_hbm.at[idx_v.at[pl.ds(n * CH, CH)]], buf.at[bn], sg)

        b = lax.rem(c, NSLOT)
        pltpu.make_async_copy(
            table_hbm.at[idx_v.at[pl.ds(0, CH)]], buf.at[0], sg).wait()
        slot = buf.at[b]
        _scale_buf(slot)
        pltpu.async_copy(slot, out_hbm.at[wid, c], so)
        return carry

    lax.fori_loop(0, NCH, step, 0, unroll=False)

    for _ in range(NSLOT):
        pltpu.make_async_copy(buf.at[0], out_hbm.at[wid, 0], so).wait()


_mesh = plsc.VectorSubcoreMesh(core_axis_name="c", subcore_axis_name="s")

_gather_scale = pl.kernel(
    _body,
    mesh=_mesh,
    out_type=jax.ShapeDtypeStruct((NW, NCH, CH, D), jnp.float32),
    scratch_types=[
        pltpu.VMEM((PER_W,), jnp.int32),
        pltpu.VMEM((NSLOT, CH, D), jnp.float32),
        pltpu.SemaphoreType.DMA,
        pltpu.SemaphoreType.DMA,
    ],
)


def kernel(x, embedding):
    out = _gather_scale(x.astype(jnp.int32), embedding)
    return out.reshape(BATCH, SEQ, D)


# speedup vs baseline: 1.0098x; 1.0098x over previous
import math

import jax
import jax.numpy as jnp
from jax import lax
from jax.experimental import pallas as pl
from jax.experimental.pallas import tpu as pltpu
from jax.experimental.pallas import tpu_sc as plsc

D = 1024
NC = 2
NS = 16
NW = NC * NS
BATCH = 4
SEQ = 4096
TOTAL = BATCH * SEQ
PER_W = TOTAL // NW
WPR = SEQ // PER_W
CH = 8
NCH = PER_W // CH
NSLOT = 15
LEAD = 13
LANES = 16
VPR = D // LANES
SCALE = math.sqrt(D)


def _scale_buf(buf):
    def row(r, carry):
        def grp(j, carry2):
            sl = pl.ds(j * LANES, LANES)
            buf[r, sl] = buf[r, sl] * SCALE
            return carry2

        return lax.fori_loop(0, VPR, grp, carry, unroll=8)

    lax.fori_loop(0, CH, row, 0, unroll=False)


def _body(x_hbm, table_hbm, out_hbm, idx_v, buf, sg, so):
    wid = lax.axis_index("s") * NC + lax.axis_index("c")
    pltpu.sync_copy(
        x_hbm.at[wid // WPR, pl.ds((wid % WPR) * PER_W, PER_W)], idx_v)

    for n in range(LEAD):
        pltpu.async_copy(
            table_hbm.at[idx_v.at[pl.ds(n * CH, CH)]], buf.at[n], sg)

    def step(c, carry):
        n = c + LEAD
        bn = lax.rem(n, NSLOT)

        @pl.when(n < NCH)
        def _():
            @pl.when(c >= NSLOT - LEAD)
            def _():
                pltpu.make_async_copy(
                    buf.at[0], out_hbm.at[wid, 0], so).wait()

            pltpu.async_copy(
                table_hbm.at[idx_v.at[pl.ds(n * CH, CH)]], buf.at[bn], sg)

        b = lax.rem(c, NSLOT)
        pltpu.make_async_copy(
            table_hbm.at[idx_v.at[pl.ds(0, CH)]], buf.at[0], sg).wait()
        slot = buf.at[b]
        _scale_buf(slot)
        pltpu.async_copy(slot, out_hbm.at[wid, c], so)
        return carry

    lax.fori_loop(0, NCH, step, 0, unroll=False)

    for _ in range(NSLOT):
        pltpu.make_async_copy(buf.at[0], out_hbm.at[wid, 0], so).wait()


_mesh = plsc.VectorSubcoreMesh(core_axis_name="c", subcore_axis_name="s")

_gather_scale = pl.kernel(
    _body,
    mesh=_mesh,
    out_type=jax.ShapeDtypeStruct((NW, NCH, CH, D), jnp.float32),
    scratch_types=[
        pltpu.VMEM((PER_W,), jnp.int32),
        pltpu.VMEM((NSLOT, CH, D), jnp.float32),
        pltpu.SemaphoreType.DMA,
        pltpu.SemaphoreType.DMA,
    ],
)


def kernel(x, embedding):
    out = _gather_scale(x.astype(jnp.int32), embedding)
    return out.reshape(BATCH, SEQ, D)


# speedup vs baseline: 1.0235x; 1.0136x over previous
import math

import jax
import jax.numpy as jnp
from jax import lax
from jax.experimental import pallas as pl
from jax.experimental.pallas import tpu as pltpu
from jax.experimental.pallas import tpu_sc as plsc

D = 1024
NC = 2
NS = 16
NW = NC * NS
BATCH = 4
SEQ = 4096
TOTAL = BATCH * SEQ
PER_W = TOTAL // NW
WPR = SEQ // PER_W
CH = 8
NCH = PER_W // CH
NSLOT = 15
LEAD = 14
LANES = 16
VPR = D // LANES
SCALE = math.sqrt(D)


def _scale_buf(buf):
    def row(r, carry):
        def grp(j, carry2):
            sl = pl.ds(j * LANES, LANES)
            buf[r, sl] = buf[r, sl] * SCALE
            return carry2

        return lax.fori_loop(0, VPR, grp, carry, unroll=8)

    lax.fori_loop(0, CH, row, 0, unroll=False)


def _body(x_hbm, table_hbm, out_hbm, idx_v, buf, sg, so):
    wid = lax.axis_index("s") * NC + lax.axis_index("c")
    pltpu.sync_copy(
        x_hbm.at[wid // WPR, pl.ds((wid % WPR) * PER_W, PER_W)], idx_v)

    for n in range(LEAD):
        pltpu.async_copy(
            table_hbm.at[idx_v.at[pl.ds(n * CH, CH)]], buf.at[n], sg)

    def step(c, carry):
        n = c + LEAD
        bn = lax.rem(n, NSLOT)

        @pl.when(n < NCH)
        def _():
            @pl.when(c >= NSLOT - LEAD)
            def _():
                pltpu.make_async_copy(
                    buf.at[0], out_hbm.at[wid, 0], so).wait()

            pltpu.async_copy(
                table_hbm.at[idx_v.at[pl.ds(n * CH, CH)]], buf.at[bn], sg)

        b = lax.rem(c, NSLOT)
        pltpu.make_async_copy(
            table_hbm.at[idx_v.at[pl.ds(0, CH)]], buf.at[0], sg).wait()
        slot = buf.at[b]
        _scale_buf(slot)
        pltpu.async_copy(slot, out_hbm.at[wid, c], so)
        return carry

    lax.fori_loop(0, NCH, step, 0, unroll=False)

    for _ in range(NSLOT):
        pltpu.make_async_copy(buf.at[0], out_hbm.at[wid, 0], so).wait()


_mesh = plsc.VectorSubcoreMesh(core_axis_name="c", subcore_axis_name="s")

_gather_scale = pl.kernel(
    _body,
    mesh=_mesh,
    out_type=jax.ShapeDtypeStruct((NW, NCH, CH, D), jnp.float32),
    scratch_types=[
        pltpu.VMEM((PER_W,), jnp.int32),
        pltpu.VMEM((NSLOT, CH, D), jnp.float32),
        pltpu.SemaphoreType.DMA,
        pltpu.SemaphoreType.DMA,
    ],
)


def kernel(x, embedding):
    out = _gather_scale(x.astype(jnp.int32), embedding)
    return out.reshape(BATCH, SEQ, D)
